# Initial kernel scaffold; baseline (speedup 1.0000x reference)
import jax
import jax.numpy as jnp
from jax.experimental import pallas as pl


def kernel(x, edge_index, W1, b1, gamma1, beta1, W2, b2, gamma2, beta2, W3, b3):
    raise NotImplementedError("write your pallas kernel here")


# speedup vs baseline: 25.6150x; 25.6150x over previous
import functools

import jax
import jax.numpy as jnp
from jax import lax
from jax.experimental import pallas as pl
from jax.experimental.pallas import tpu as pltpu
from jax.experimental.pallas import tpu_sc as plsc

_N = 10000
_NP = 10240
_E = 320000
_D = 128
_DO = 16
_EPS = 1e-5
_NC = 2
_NS = 16
_TILES = _NC * _NS
_CHUNK = 128
_NCHUNKS = _E // _CHUNK
_CH_BASE = _NCHUNKS // _TILES
_CH_EXTRA = _NCHUNKS - _CH_BASE * _TILES
_RPT = _NP // _NS
_NB = 2


def _fill(buf, d, val):
    npr = d // 16

    def f(i, _):
        r = i // npr
        col = (i % npr) * 16
        buf[r, pl.ds(col, 16)] = jnp.full((16,), val, jnp.float32)
        return 0

    lax.fori_loop(0, _CHUNK * npr, f, 0)


@functools.cache
def _make_sc_scatter(d, with_table):
    mesh = plsc.VectorSubcoreMesh(core_axis_name="c", subcore_axis_name="s")

    def body(*refs):
        if with_table:
            (src_hbm, dst_hbm, table, out, acc, idx_s, idx_d, rows,
             sem_i0, sem_i1, sem_g0, sem_g1, sem_wb) = refs
        else:
            (src_hbm, dst_hbm, out, acc, idx_s, idx_d, rows,
             sem_i0, sem_i1, sem_g0, sem_g1, sem_wb) = refs
        sem_i = (sem_i0, sem_i1)
        sem_g = (sem_g0, sem_g1)
        c = lax.axis_index("c")
        s = lax.axis_index("s")
        w = c * _NS + s

        _fill(rows.at[0], d, 0.0)
        r0 = s * _RPT
        for off in range(0, _RPT, _CHUNK):
            pltpu.async_copy(rows.at[0], acc.at[pl.ds(r0 + off, _CHUNK)], sem_wb)
        for off in range(0, _RPT, _CHUNK):
            pltpu.make_async_copy(rows.at[0], acc.at[pl.ds(r0 + off, _CHUNK)],
                                  sem_wb).wait()
        if not with_table:
            _fill(rows.at[0], d, 1.0)
            _fill(rows.at[1], d, 1.0)
        plsc.subcore_barrier()

        start = _CH_BASE * w + jnp.minimum(w, _CH_EXTRA)
        n_w = _CH_BASE + jnp.where(w < _CH_EXTRA, 1, 0)

        def start_idx(b, g):
            @pl.when(g < n_w)
            def _():
                base = (start + g) * _CHUNK
                pltpu.async_copy(dst_hbm.at[pl.ds(base, _CHUNK)], idx_d.at[b],
                                 sem_i[b])
                if with_table:
                    pltpu.async_copy(src_hbm.at[pl.ds(base, _CHUNK)],
                                     idx_s.at[b], sem_i[b])

        def wait_idx(b, g):
            @pl.when(g < n_w)
            def _():
                base = (start + g) * _CHUNK
                pltpu.make_async_copy(dst_hbm.at[pl.ds(base, _CHUNK)],
                                      idx_d.at[b], sem_i[b]).wait()
                if with_table:
                    pltpu.make_async_copy(src_hbm.at[pl.ds(base, _CHUNK)],
                                          idx_s.at[b], sem_i[b]).wait()

        def start_gather(b, g):
            if with_table:
                @pl.when(g < n_w)
                def _():
                    pltpu.async_copy(table.at[idx_s.at[b]], rows.at[b], sem_g[b])

        def wait_gather(b, g):
            if with_table:
                @pl.when(g < n_w)
                def _():
                    pltpu.make_async_copy(table.at[idx_s.at[b]], rows.at[b],
                                          sem_g[b]).wait()

        def scatter(b, g):
            @pl.when(g < n_w)
            def _():
                pltpu.sync_copy(rows.at[b], acc.at[idx_d.at[b]], add=True)

        start_idx(0, 0)
        start_idx(1, 1)
        wait_idx(0, 0)
        start_gather(0, 0)

        def step(j, _):
            g0 = j * _NB

            def stage(b, g):
                wait_idx((b + 1) % _NB, g + 1)
                start_gather((b + 1) % _NB, g + 1)
                wait_gather(b, g)
                scatter(b, g)
                start_idx(b, g + 2)

            stage(0, g0)
            stage(1, g0 + 1)
            return 0

        lax.fori_loop(0, (_CH_BASE + 1 + _NB - 1) // _NB, step, 0)
        plsc.subcore_barrier()

        for off in range(0, _RPT, _CHUNK):
            pltpu.async_copy(acc.at[pl.ds(r0 + off, _CHUNK)],
                             out.at[c, pl.ds(r0 + off, _CHUNK)], sem_wb)
        for off in range(0, _RPT, _CHUNK):
            pltpu.make_async_copy(acc.at[pl.ds(r0 + off, _CHUNK)],
                                  out.at[c, pl.ds(r0 + off, _CHUNK)],
                                  sem_wb).wait()

    scratch = [
        pltpu.VMEM_SHARED((_NP, d), jnp.float32),
        pltpu.VMEM((_NB, _CHUNK), jnp.int32),
        pltpu.VMEM((_NB, _CHUNK), jnp.int32),
        pltpu.VMEM((_NB, _CHUNK, d), jnp.float32),
        pltpu.SemaphoreType.DMA,
        pltpu.SemaphoreType.DMA,
        pltpu.SemaphoreType.DMA,
        pltpu.SemaphoreType.DMA,
        pltpu.SemaphoreType.DMA,
    ]

    return pl.kernel(
        body,
        out_type=jax.ShapeDtypeStruct((_NC, _NP, d), jnp.float32),
        mesh=mesh,
        scratch_types=scratch,
        compiler_params=pltpu.CompilerParams(use_tc_tiling_on_sc=(d == _D)),
    )


def _dot(a, b):
    return jnp.dot(a, b, preferred_element_type=jnp.float32,
                   precision=lax.Precision.HIGHEST)


def _tc1_body(degp_ref, x_ref, w1_ref, dinvb_ref, g1_ref):
    deg = (degp_ref[0][0:_N, 0:1] + degp_ref[1][0:_N, 0:1]) + 1.0
    dinvb = jnp.broadcast_to(lax.rsqrt(deg), (_N, _D))
    dinvb_ref[...] = dinvb
    g1_ref[...] = dinvb * _dot(x_ref[...], w1_ref[...])


def _tc_mid_body(s_ref, g_ref, dinvb_ref, b_ref, gamma_ref, beta_ref, w_ref,
                 out_ref, out_w):
    dinvb = dinvb_ref[...]
    z = dinvb * (s_ref[0][0:_N] + s_ref[1][0:_N] + g_ref[...]) + b_ref[...]
    mu = jnp.mean(z, axis=0, keepdims=True)
    var = jnp.mean((z - mu) ** 2, axis=0, keepdims=True)
    zn = gamma_ref[...] * (z - mu) / jnp.sqrt(var + _EPS) + beta_ref[...]
    a = jnp.maximum(zn, 0.0)
    out_ref[...] = dinvb[:, 0:out_w] * _dot(a, w_ref[...])


def _tc4_body(s_ref, g_ref, dinvb_ref, b3_ref, out_ref):
    z16 = (dinvb_ref[:, 0:_DO] * (s_ref[0][0:_N] + s_ref[1][0:_N] + g_ref[...])
           + b3_ref[...])
    col = lax.broadcasted_iota(jnp.int32, (_N, _DO), 1)
    z = jnp.where(col < 5, z16, -1e30)
    m = jnp.max(z, axis=1, keepdims=True)
    lse = jnp.log(jnp.sum(jnp.exp(z - m), axis=1, keepdims=True)) + m
    out_ref[...] = (z - lse)[:, 0:5]


_f32 = lambda shape: jax.ShapeDtypeStruct(shape, jnp.float32)

_tc1 = pl.pallas_call(_tc1_body, out_shape=(_f32((_N, _D)), _f32((_N, _D))))
_tc2 = pl.pallas_call(functools.partial(_tc_mid_body, out_w=_D),
                      out_shape=_f32((_N, _D)))
_tc3 = pl.pallas_call(functools.partial(_tc_mid_body, out_w=_DO),
                      out_shape=_f32((_N, _DO)))
_tc4 = pl.pallas_call(_tc4_body, out_shape=_f32((_N, 5)))


def kernel(x, edge_index, W1, b1, gamma1, beta1, W2, b2, gamma2, beta2, W3, b3):
    src, dst = edge_index[0], edge_index[1]
    sc_degree = _make_sc_scatter(_DO, with_table=False)
    sc_scat128 = _make_sc_scatter(_D, with_table=True)
    sc_scat16 = _make_sc_scatter(_DO, with_table=True)
    degp = sc_degree(src, dst)
    dinvb, g1 = _tc1(degp, x, W1)
    s1 = sc_scat128(src, dst, g1)
    g2 = _tc2(s1, g1, dinvb, b1.reshape(1, -1), gamma1.reshape(1, -1),
              beta1.reshape(1, -1), W2)
    s2 = sc_scat128(src, dst, g2)
    W3p = jnp.pad(W3, ((0, 0), (0, _DO - 5)))
    g3 = _tc3(s2, g2, dinvb, b2.reshape(1, -1), gamma2.reshape(1, -1),
              beta2.reshape(1, -1), W3p)
    s3 = sc_scat16(src, dst, g3)
    b3p = jnp.pad(b3, (0, _DO - 5)).reshape(1, -1)
    return _tc4(s3, g3, dinvb, b3p)


# speedup vs baseline: 32.6212x; 1.2735x over previous
import functools

import jax
import jax.numpy as jnp
from jax import lax
from jax.experimental import pallas as pl
from jax.experimental.pallas import tpu as pltpu
from jax.experimental.pallas import tpu_sc as plsc

_N = 10000
_NP = 10240
_E = 320000
_D = 128
_DO = 16
_EPS = 1e-5
_NC = 2
_NS = 16
_TILES = _NC * _NS
_CHUNK = 128
_NCHUNKS = _E // _CHUNK
_CH_BASE = _NCHUNKS // _TILES
_CH_EXTRA = _NCHUNKS - _CH_BASE * _TILES
_RPT = _NP // _NS
_NB = 2


def _fill(buf, d, val):
    npr = d // 16

    def f(i, _):
        r = i // npr
        col = (i % npr) * 16
        buf[r, pl.ds(col, 16)] = jnp.full((16,), val, jnp.float32)
        return 0

    lax.fori_loop(0, _CHUNK * npr, f, 0)


@functools.cache
def _make_sc_scatter(d, with_table, k):
    assert _CH_BASE % k == 0
    n_super = _CH_BASE // k
    mesh = plsc.VectorSubcoreMesh(core_axis_name="c", subcore_axis_name="s")

    def body(*refs):
        if with_table:
            (src_hbm, dst_hbm, table, out, acc, idx_s, idx_d, rows,
             sem_i0, sem_i1, sem_i2, sem_i3,
             sem_g0, sem_g1, sem_s0, sem_s1, sem_wb) = refs
            ones_buf = None
        else:
            (src_hbm, dst_hbm, out, acc, idx_s, idx_d, ones_buf,
             sem_i0, sem_i1, sem_i2, sem_i3,
             sem_g0, sem_g1, sem_s0, sem_s1, sem_wb) = refs
            rows = None
        sem_i = (sem_i0, sem_i1, sem_i2, sem_i3)
        sem_g = (sem_g0, sem_g1)
        sem_s = (sem_s0, sem_s1)
        c = lax.axis_index("c")
        s = lax.axis_index("s")
        w = c * _NS + s

        zbuf = rows.at[0] if with_table else ones_buf
        _fill(zbuf, d, 0.0)
        r0 = s * _RPT
        for off in range(0, _RPT, _CHUNK):
            pltpu.async_copy(zbuf, acc.at[pl.ds(r0 + off, _CHUNK)], sem_wb)
        for off in range(0, _RPT, _CHUNK):
            pltpu.make_async_copy(zbuf, acc.at[pl.ds(r0 + off, _CHUNK)],
                                  sem_wb).wait()
        if not with_table:
            _fill(ones_buf, d, 1.0)
        plsc.subcore_barrier()

        start = _CH_BASE * w + jnp.minimum(w, _CH_EXTRA)

        def cbase(g, j):
            return (start + jnp.minimum(g * k + j, _CH_BASE - 1)) * _CHUNK

        def start_idx(q, g):
            for j in range(k):
                base = cbase(g, j)
                pltpu.async_copy(dst_hbm.at[pl.ds(base, _CHUNK)],
                                 idx_d.at[q * k + j], sem_i[q])
                if with_table:
                    pltpu.async_copy(src_hbm.at[pl.ds(base, _CHUNK)],
                                     idx_s.at[q * k + j], sem_i[q])

        def wait_idx(q, g):
            for j in range(k):
                base = cbase(g, j)
                pltpu.make_async_copy(dst_hbm.at[pl.ds(base, _CHUNK)],
                                      idx_d.at[q * k + j], sem_i[q]).wait()
                if with_table:
                    pltpu.make_async_copy(src_hbm.at[pl.ds(base, _CHUNK)],
                                          idx_s.at[q * k + j], sem_i[q]).wait()

        def start_gather(b, q):
            if with_table:
                for j in range(k):
                    pltpu.async_copy(table.at[idx_s.at[q * k + j]],
                                     rows.at[b * k + j], sem_g[b])

        def wait_gather(b, q):
            if with_table:
                for j in range(k):
                    pltpu.make_async_copy(table.at[idx_s.at[q * k + j]],
                                          rows.at[b * k + j], sem_g[b]).wait()

        def start_scatter(b, q):
            for j in range(k):
                src = rows.at[b * k + j] if with_table else ones_buf
                pltpu.async_copy(src, acc.at[idx_d.at[q * k + j]], sem_s[b],
                                 add=True)

        def wait_scatter(b, q):
            for j in range(k):
                src = rows.at[b * k + j] if with_table else ones_buf
                pltpu.make_async_copy(src, acc.at[idx_d.at[q * k + j]],
                                      sem_s[b]).wait()

        def stage(b, q, g, first=False):
            wait_idx((q + 1) % 4, g + 1)
            if not first:
                wait_scatter(1 - b, (q - 1) % 4)
            start_idx((q + 3) % 4, g + 3)
            start_gather(1 - b, (q + 1) % 4)
            wait_gather(b, q)
            start_scatter(b, q)

        start_idx(0, 0)
        start_idx(1, 1)
        start_idx(2, 2)
        wait_idx(0, 0)
        start_gather(0, 0)

        stage(0, 0, 0, first=True)
        stage(1, 1, 1)

        n_loop = (n_super - 2) // 4

        def step4(i, _):
            g0 = 2 + i * 4
            stage(0, 2, g0)
            stage(1, 3, g0 + 1)
            stage(0, 0, g0 + 2)
            stage(1, 1, g0 + 3)
            return 0

        lax.fori_loop(0, n_loop, step4, 0)
        for g in range(2 + 4 * n_loop, n_super):
            stage(g % 2, g % 4, g)

        bf = (n_super - 1) % 2
        wait_idx((n_super + 1) % 4, n_super + 1)
        wait_idx((n_super + 2) % 4, n_super + 2)
        wait_gather(1 - bf, n_super % 4)
        wait_scatter(bf, (n_super - 1) % 4)

        @pl.when(w < _CH_EXTRA)
        def _():
            base = (start + _CH_BASE) * _CHUNK
            pltpu.sync_copy(dst_hbm.at[pl.ds(base, _CHUNK)], idx_d.at[0])
            if with_table:
                pltpu.sync_copy(src_hbm.at[pl.ds(base, _CHUNK)], idx_s.at[0])
                pltpu.async_copy(table.at[idx_s.at[0]], rows.at[0],
                                 sem_g[0]).wait()
                pltpu.sync_copy(rows.at[0], acc.at[idx_d.at[0]], add=True)
            else:
                pltpu.sync_copy(ones_buf, acc.at[idx_d.at[0]], add=True)

        plsc.subcore_barrier()

        for off in range(0, _RPT, _CHUNK):
            pltpu.async_copy(acc.at[pl.ds(r0 + off, _CHUNK)],
                             out.at[c, pl.ds(r0 + off, _CHUNK)], sem_wb)
        for off in range(0, _RPT, _CHUNK):
            pltpu.make_async_copy(acc.at[pl.ds(r0 + off, _CHUNK)],
                                  out.at[c, pl.ds(r0 + off, _CHUNK)],
                                  sem_wb).wait()

    scratch = [
        pltpu.VMEM_SHARED((_NP, d), jnp.float32),
        pltpu.VMEM((4 * k, _CHUNK), jnp.int32),
        pltpu.VMEM((4 * k, _CHUNK), jnp.int32),
    ]
    if with_table:
        scratch.append(pltpu.VMEM((_NB * k, _CHUNK, d), jnp.float32))
    else:
        scratch.append(pltpu.VMEM((_CHUNK, d), jnp.float32))
    scratch += [pltpu.SemaphoreType.DMA] * 4
    scratch += [pltpu.SemaphoreType.DMA] * 2
    scratch += [pltpu.SemaphoreType.DMA] * 2
    scratch += [pltpu.SemaphoreType.DMA]

    return pl.kernel(
        body,
        out_type=jax.ShapeDtypeStruct((_NC, _NP, d), jnp.float32),
        mesh=mesh,
        scratch_types=scratch,
        compiler_params=pltpu.CompilerParams(use_tc_tiling_on_sc=(d == _D)),
    )


def _dot(a, b):
    return jnp.dot(a, b, preferred_element_type=jnp.float32,
                   precision=lax.Precision.HIGHEST)


def _tc1_body(degp_ref, x_ref, w1_ref, dinvb_ref, g1_ref):
    deg = (degp_ref[0][0:_N, 0:1] + degp_ref[1][0:_N, 0:1]) + 1.0
    dinvb = jnp.broadcast_to(lax.rsqrt(deg), (_N, _D))
    dinvb_ref[...] = dinvb
    g1_ref[...] = dinvb * _dot(x_ref[...], w1_ref[...])


def _tc_mid_body(s_ref, g_ref, dinvb_ref, b_ref, gamma_ref, beta_ref, w_ref,
                 out_ref, out_w):
    dinvb = dinvb_ref[...]
    z = dinvb * (s_ref[0][0:_N] + s_ref[1][0:_N] + g_ref[...]) + b_ref[...]
    mu = jnp.mean(z, axis=0, keepdims=True)
    var = jnp.mean((z - mu) ** 2, axis=0, keepdims=True)
    zn = gamma_ref[...] * (z - mu) / jnp.sqrt(var + _EPS) + beta_ref[...]
    a = jnp.maximum(zn, 0.0)
    out_ref[...] = dinvb[:, 0:out_w] * _dot(a, w_ref[...])


def _tc4_body(s_ref, g_ref, dinvb_ref, b3_ref, out_ref):
    z16 = (dinvb_ref[:, 0:_DO] * (s_ref[0][0:_N] + s_ref[1][0:_N] + g_ref[...])
           + b3_ref[...])
    col = lax.broadcasted_iota(jnp.int32, (_N, _DO), 1)
    z = jnp.where(col < 5, z16, -1e30)
    m = jnp.max(z, axis=1, keepdims=True)
    lse = jnp.log(jnp.sum(jnp.exp(z - m), axis=1, keepdims=True)) + m
    out_ref[...] = (z - lse)[:, 0:5]


_f32 = lambda shape: jax.ShapeDtypeStruct(shape, jnp.float32)

_tc1 = pl.pallas_call(_tc1_body, out_shape=(_f32((_N, _D)), _f32((_N, _D))))
_tc2 = pl.pallas_call(functools.partial(_tc_mid_body, out_w=_D),
                      out_shape=_f32((_N, _D)))
_tc3 = pl.pallas_call(functools.partial(_tc_mid_body, out_w=_DO),
                      out_shape=_f32((_N, _DO)))
_tc4 = pl.pallas_call(_tc4_body, out_shape=_f32((_N, 5)))


def kernel(x, edge_index, W1, b1, gamma1, beta1, W2, b2, gamma2, beta2, W3, b3):
    src, dst = edge_index[0], edge_index[1]
    sc_degree = _make_sc_scatter(_DO, with_table=False, k=6)
    sc_scat128 = _make_sc_scatter(_D, with_table=True, k=1)
    sc_scat16 = _make_sc_scatter(_DO, with_table=True, k=6)
    degp = sc_degree(src, dst)
    dinvb, g1 = _tc1(degp, x, W1)
    s1 = sc_scat128(src, dst, g1)
    g2 = _tc2(s1, g1, dinvb, b1.reshape(1, -1), gamma1.reshape(1, -1),
              beta1.reshape(1, -1), W2)
    s2 = sc_scat128(src, dst, g2)
    W3p = jnp.pad(W3, ((0, 0), (0, _DO - 5)))
    g3 = _tc3(s2, g2, dinvb, b2.reshape(1, -1), gamma2.reshape(1, -1),
              beta2.reshape(1, -1), W3p)
    s3 = sc_scat16(src, dst, g3)
    b3p = jnp.pad(b3, (0, _DO - 5)).reshape(1, -1)
    return _tc4(s3, g3, dinvb, b3p)


# speedup vs baseline: 34.5673x; 1.0597x over previous
import functools

import jax
import jax.numpy as jnp
from jax import lax
from jax.experimental import pallas as pl
from jax.experimental.pallas import tpu as pltpu
from jax.experimental.pallas import tpu_sc as plsc

_N = 10000
_NP = 10240
_E = 320000
_D = 128
_DO = 16
_EPS = 1e-5
_NC = 2
_NS = 16
_TILES = _NC * _NS
_CHUNK = 128
_NCHUNKS = _E // _CHUNK
_CH_BASE = _NCHUNKS // _TILES
_CH_EXTRA = _NCHUNKS - _CH_BASE * _TILES
_RPT = _NP // _NS
_NB = 2


def _fill(buf, d, val):
    v = jnp.full((16,), val, jnp.float32)

    def f(r, _):
        for j in range(d // 16):
            buf[r, pl.ds(j * 16, 16)] = v
        return 0

    lax.fori_loop(0, _CHUNK, f, 0)


@functools.cache
def _make_sc_scatter(d, with_table, k):
    assert _CH_BASE % k == 0
    n_super = _CH_BASE // k
    mesh = plsc.VectorSubcoreMesh(core_axis_name="c", subcore_axis_name="s")

    def body(*refs):
        if with_table:
            (src_hbm, dst_hbm, table, out, acc, idx_s, idx_d, rows,
             sem_i0, sem_i1, sem_i2, sem_i3,
             sem_g0, sem_g1, sem_s0, sem_s1, sem_wb) = refs
            ones_buf = None
        else:
            (src_hbm, dst_hbm, out, acc, idx_s, idx_d, ones_buf,
             sem_i0, sem_i1, sem_i2, sem_i3,
             sem_g0, sem_g1, sem_s0, sem_s1, sem_wb) = refs
            rows = None
        sem_i = (sem_i0, sem_i1, sem_i2, sem_i3)
        sem_g = (sem_g0, sem_g1)
        sem_s = (sem_s0, sem_s1)
        c = lax.axis_index("c")
        s = lax.axis_index("s")
        w = c * _NS + s

        zbuf = rows.at[k] if with_table else ones_buf
        _fill(zbuf, d, 0.0)
        r0 = s * _RPT
        for off in range(0, _RPT, _CHUNK):
            pltpu.async_copy(zbuf, acc.at[pl.ds(r0 + off, _CHUNK)], sem_wb)

        start = _CH_BASE * w + jnp.minimum(w, _CH_EXTRA)

        def cbase(g, j):
            return (start + jnp.minimum(g * k + j, _CH_BASE - 1)) * _CHUNK

        def start_idx(q, g):
            for j in range(k):
                base = cbase(g, j)
                pltpu.async_copy(dst_hbm.at[pl.ds(base, _CHUNK)],
                                 idx_d.at[q * k + j], sem_i[q])
                if with_table:
                    pltpu.async_copy(src_hbm.at[pl.ds(base, _CHUNK)],
                                     idx_s.at[q * k + j], sem_i[q])

        def wait_idx(q, g):
            for j in range(k):
                base = cbase(g, j)
                pltpu.make_async_copy(dst_hbm.at[pl.ds(base, _CHUNK)],
                                      idx_d.at[q * k + j], sem_i[q]).wait()
                if with_table:
                    pltpu.make_async_copy(src_hbm.at[pl.ds(base, _CHUNK)],
                                          idx_s.at[q * k + j], sem_i[q]).wait()

        def start_gather(b, q):
            if with_table:
                for j in range(k):
                    pltpu.async_copy(table.at[idx_s.at[q * k + j]],
                                     rows.at[b * k + j], sem_g[b])

        def wait_gather(b, q):
            if with_table:
                for j in range(k):
                    pltpu.make_async_copy(table.at[idx_s.at[q * k + j]],
                                          rows.at[b * k + j], sem_g[b]).wait()

        def start_scatter(b, q):
            for j in range(k):
                src = rows.at[b * k + j] if with_table else ones_buf
                pltpu.async_copy(src, acc.at[idx_d.at[q * k + j]], sem_s[b],
                                 add=True)

        def wait_scatter(b, q):
            for j in range(k):
                src = rows.at[b * k + j] if with_table else ones_buf
                pltpu.make_async_copy(src, acc.at[idx_d.at[q * k + j]],
                                      sem_s[b]).wait()

        def stage(b, q, g, first=False):
            wait_idx((q + 1) % 4, g + 1)
            if not first:
                wait_scatter(1 - b, (q - 1) % 4)
            start_idx((q + 3) % 4, g + 3)
            start_gather(1 - b, (q + 1) % 4)
            wait_gather(b, q)
            start_scatter(b, q)

        start_idx(0, 0)
        start_idx(1, 1)
        start_idx(2, 2)
        wait_idx(0, 0)
        start_gather(0, 0)
        for off in range(0, _RPT, _CHUNK):
            pltpu.make_async_copy(zbuf, acc.at[pl.ds(r0 + off, _CHUNK)],
                                  sem_wb).wait()
        if not with_table:
            _fill(ones_buf, d, 1.0)
        plsc.subcore_barrier()

        stage(0, 0, 0, first=True)
        stage(1, 1, 1)

        n_loop = (n_super - 2) // 4

        def step4(i, _):
            g0 = 2 + i * 4
            stage(0, 2, g0)
            stage(1, 3, g0 + 1)
            stage(0, 0, g0 + 2)
            stage(1, 1, g0 + 3)
            return 0

        lax.fori_loop(0, n_loop, step4, 0)
        for g in range(2 + 4 * n_loop, n_super):
            stage(g % 2, g % 4, g)

        bf = (n_super - 1) % 2
        wait_idx((n_super + 1) % 4, n_super + 1)
        wait_idx((n_super + 2) % 4, n_super + 2)
        wait_gather(1 - bf, n_super % 4)
        wait_scatter(bf, (n_super - 1) % 4)

        @pl.when(w < _CH_EXTRA)
        def _():
            base = (start + _CH_BASE) * _CHUNK
            pltpu.sync_copy(dst_hbm.at[pl.ds(base, _CHUNK)], idx_d.at[0])
            if with_table:
                pltpu.sync_copy(src_hbm.at[pl.ds(base, _CHUNK)], idx_s.at[0])
                pltpu.async_copy(table.at[idx_s.at[0]], rows.at[0],
                                 sem_g[0]).wait()
                pltpu.sync_copy(rows.at[0], acc.at[idx_d.at[0]], add=True)
            else:
                pltpu.sync_copy(ones_buf, acc.at[idx_d.at[0]], add=True)

        plsc.subcore_barrier()

        def wb(sizes):
            for off, cnt in sizes:
                pltpu.async_copy(acc.at[pl.ds(r0 + off, cnt)],
                                 out.at[c, pl.ds(r0 + off, cnt)], sem_wb)
            for off, cnt in sizes:
                pltpu.make_async_copy(acc.at[pl.ds(r0 + off, cnt)],
                                      out.at[c, pl.ds(r0 + off, cnt)],
                                      sem_wb).wait()

        @pl.when(s < _NS - 1)
        def _():
            wb([(off, _CHUNK) for off in range(0, _RPT, _CHUNK)])

        @pl.when(s == _NS - 1)
        def _():
            wb([(0, _CHUNK), (_CHUNK, _CHUNK), (2 * _CHUNK, _CHUNK),
                (3 * _CHUNK, _N - (_NS - 1) * _RPT - 3 * _CHUNK)])

    scratch = [
        pltpu.VMEM_SHARED((_NP, d), jnp.float32),
        pltpu.VMEM((4 * k, _CHUNK), jnp.int32),
        pltpu.VMEM((4 * k, _CHUNK), jnp.int32),
    ]
    if with_table:
        scratch.append(pltpu.VMEM((_NB * k, _CHUNK, d), jnp.float32))
    else:
        scratch.append(pltpu.VMEM((_CHUNK, d), jnp.float32))
    scratch += [pltpu.SemaphoreType.DMA] * 4
    scratch += [pltpu.SemaphoreType.DMA] * 2
    scratch += [pltpu.SemaphoreType.DMA] * 2
    scratch += [pltpu.SemaphoreType.DMA]

    return pl.kernel(
        body,
        out_type=jax.ShapeDtypeStruct((_NC, _N, d), jnp.float32),
        mesh=mesh,
        scratch_types=scratch,
        compiler_params=pltpu.CompilerParams(use_tc_tiling_on_sc=(d == _D)),
    )


def _dot(a, b):
    return jnp.dot(a, b, preferred_element_type=jnp.float32)


def _tc1_body(degp_ref, x_ref, w1_ref, dinvb_ref, g1_ref):
    deg = (degp_ref[0][0:_N, 0:1] + degp_ref[1][0:_N, 0:1]) + 1.0
    dinvb = jnp.broadcast_to(lax.rsqrt(deg), (_N, _D))
    dinvb_ref[...] = dinvb
    g1_ref[...] = dinvb * _dot(x_ref[...], w1_ref[...])


def _tc_mid_body(s_ref, g_ref, dinvb_ref, b_ref, gamma_ref, beta_ref, w_ref,
                 out_ref, out_w):
    dinvb = dinvb_ref[...]
    z = dinvb * (s_ref[0][0:_N] + s_ref[1][0:_N] + g_ref[...]) + b_ref[...]
    mu = jnp.mean(z, axis=0, keepdims=True)
    var = jnp.mean((z - mu) ** 2, axis=0, keepdims=True)
    zn = gamma_ref[...] * (z - mu) / jnp.sqrt(var + _EPS) + beta_ref[...]
    a = jnp.maximum(zn, 0.0)
    out_ref[...] = dinvb[:, 0:out_w] * _dot(a, w_ref[...])


def _tc4_body(s_ref, g_ref, dinvb_ref, b3_ref, out_ref):
    z16 = (dinvb_ref[:, 0:_DO] * (s_ref[0][0:_N] + s_ref[1][0:_N] + g_ref[...])
           + b3_ref[...])
    col = lax.broadcasted_iota(jnp.int32, (_N, _DO), 1)
    z = jnp.where(col < 5, z16, -1e30)
    m = jnp.max(z, axis=1, keepdims=True)
    lse = jnp.log(jnp.sum(jnp.exp(z - m), axis=1, keepdims=True)) + m
    out_ref[...] = (z - lse)[:, 0:5]


_f32 = lambda shape: jax.ShapeDtypeStruct(shape, jnp.float32)

_tc1 = pl.pallas_call(_tc1_body, out_shape=(_f32((_N, _D)), _f32((_N, _D))))
_tc2 = pl.pallas_call(functools.partial(_tc_mid_body, out_w=_D),
                      out_shape=_f32((_N, _D)))
_tc3 = pl.pallas_call(functools.partial(_tc_mid_body, out_w=_DO),
                      out_shape=_f32((_N, _DO)))
_tc4 = pl.pallas_call(_tc4_body, out_shape=_f32((_N, 5)))


def kernel(x, edge_index, W1, b1, gamma1, beta1, W2, b2, gamma2, beta2, W3, b3):
    src, dst = edge_index[0], edge_index[1]
    sc_degree = _make_sc_scatter(_DO, with_table=False, k=6)
    sc_scat128 = _make_sc_scatter(_D, with_table=True, k=1)
    sc_scat16 = _make_sc_scatter(_DO, with_table=True, k=6)
    degp = sc_degree(src, dst)
    dinvb, g1 = _tc1(degp, x, W1)
    s1 = sc_scat128(src, dst, g1)
    g2 = _tc2(s1, g1, dinvb, b1.reshape(1, -1), gamma1.reshape(1, -1),
              beta1.reshape(1, -1), W2)
    s2 = sc_scat128(src, dst, g2)
    W3p = jnp.pad(W3, ((0, 0), (0, _DO - 5)))
    g3 = _tc3(s2, g2, dinvb, b2.reshape(1, -1), gamma2.reshape(1, -1),
              beta2.reshape(1, -1), W3p)
    s3 = sc_scat16(src, dst, g3)
    b3p = jnp.pad(b3, (0, _DO - 5)).reshape(1, -1)
    return _tc4(s3, g3, dinvb, b3p)
